# baseline (speedup 1.0000x reference)
import functools

import jax
import jax.numpy as jnp
from jax import lax
from jax.experimental import pallas as pl
from jax.experimental.pallas import tpu as pltpu
from jax.experimental.pallas import tpu_sc as plsc

B = 1024
N_CHANS = 64
WINDOW = 1000
N_OUT = 4
E = 16
FEAT = 512

WB = 8
NSTEP = WINDOW // WB


def _tc_body(xT_ref, Wbb_ref, bbb_ref, Wall_ref, allh_ref, acc_ref):
    i = pl.program_id(0)

    @pl.when(i == 0)
    def _():
        acc_ref[...] = jnp.zeros_like(acc_ref)

    acc_ref[...] += jnp.sum(xT_ref[...], axis=1)

    @pl.when(i == NSTEP - 1)
    def _():
        m = acc_ref[...] * (1.0 / WINDOW)
        dn = (((0,), (0,)), ((), ()))
        feats = jax.lax.dot_general(m, Wbb_ref[...], dn,
                                    preferred_element_type=jnp.float32)
        feats = jnp.maximum(feats + bbb_ref[...], 0.0)
        Wv = Wall_ref[...].reshape(E * N_OUT, FEAT)
        dn_t = (((1,), (1,)), ((), ()))
        allh = jax.lax.dot_general(feats, Wv, dn_t,
                                   preferred_element_type=jnp.float32)
        allh_ref[...] = jnp.concatenate(
            [allh, jnp.zeros_like(allh)], axis=1)


SC_STRIPE = 128


ROW = 2 * E * N_OUT


def _sc_route_body(allh_hbm, sid_hbm, bh_hbm, out_hbm, allh_v, sid_v, bh_v,
                   out_v, nc):
    wid = lax.axis_index("s") * nc + lax.axis_index("c")
    g = wid // N_OUT
    o = wid % N_OUT
    base = g * SC_STRIPE
    pltpu.sync_copy(allh_hbm.at[pl.ds(base * ROW, SC_STRIPE * ROW)], allh_v)
    pltpu.sync_copy(sid_hbm.at[pl.ds(base, SC_STRIPE)], sid_v)
    pltpu.sync_copy(bh_hbm, bh_v)
    lanes = jax.lax.iota(jnp.int32, 16)
    for h in range(SC_STRIPE // 16):
        sidvec = sid_v[pl.ds(h * 16, 16)]
        idx = (lanes + h * 16) * ROW + sidvec * N_OUT + o
        val = plsc.load_gather(allh_v, [idx])
        bias = plsc.load_gather(bh_v, [sidvec * N_OUT + o])
        out_v[pl.ds(h * 16, 16)] = val + bias
    pltpu.sync_copy(out_v,
                    out_hbm.at[pl.ds(g * (N_OUT * SC_STRIPE) + o * SC_STRIPE,
                                     SC_STRIPE)])


@jax.jit
def kernel(x, subject_ids, W_bb, b_bb, W_heads, b_heads):
    xT = jnp.transpose(x, (1, 2, 0))
    sid = subject_ids.astype(jnp.int32)
    W_v = W_heads.transpose(0, 2, 1)
    bh_flat = b_heads.reshape(E * N_OUT)
    bbb = b_bb.reshape(1, FEAT)

    allh = pl.pallas_call(
        _tc_body,
        grid=(NSTEP,),
        in_specs=[
            pl.BlockSpec((N_CHANS, WB, B), lambda i: (0, i, 0)),
            pl.BlockSpec((N_CHANS, FEAT), lambda i: (0, 0)),
            pl.BlockSpec((1, FEAT), lambda i: (0, 0)),
            pl.BlockSpec((E, N_OUT, FEAT), lambda i: (0, 0, 0)),
        ],
        out_specs=pl.BlockSpec((B, ROW), lambda i: (0, 0)),
        out_shape=jax.ShapeDtypeStruct((B, ROW), jnp.float32),
        scratch_shapes=[pltpu.VMEM((N_CHANS, B), jnp.float32)],
    )(xT, W_bb, bbb, W_v)
    allh_flat = allh.reshape(B * ROW)

    info = plsc.get_sparse_core_info()
    nc = info.num_cores
    mesh = plsc.VectorSubcoreMesh(core_axis_name="c", subcore_axis_name="s")
    sc_route = pl.kernel(
        functools.partial(_sc_route_body, nc=nc),
        mesh=mesh,
        compiler_params=pltpu.CompilerParams(use_tc_tiling_on_sc=False,
                                             needs_layout_passes=False,
                                             skip_device_barrier=True),
        out_type=jax.ShapeDtypeStruct((N_OUT * B,), jnp.float32),
        scratch_types=[
            pltpu.VMEM((SC_STRIPE * ROW,), jnp.float32),
            pltpu.VMEM((SC_STRIPE,), jnp.int32),
            pltpu.VMEM((E * N_OUT,), jnp.float32),
            pltpu.VMEM((SC_STRIPE,), jnp.float32),
        ],
    )
    out_flat = sc_route(allh_flat, sid, bh_flat)
    return (out_flat.reshape(B // SC_STRIPE, N_OUT, SC_STRIPE)
            .transpose(1, 0, 2).reshape(N_OUT, B).T)


# speedup vs baseline: 1.5934x; 1.5934x over previous
import functools

import jax
import jax.numpy as jnp
from jax import lax
from jax.experimental import pallas as pl
from jax.experimental.pallas import tpu as pltpu
from jax.experimental.pallas import tpu_sc as plsc

B = 1024
N_CHANS = 64
WINDOW = 1000
N_OUT = 4
E = 16
FEAT = 512

WB = 40
NSTEP = WINDOW // WB


def _tc_body(xT_ref, Wbb_ref, bbb_ref, Wall_ref, allh_ref, acc_ref):
    i = pl.program_id(0)

    @pl.when(i == 0)
    def _():
        acc_ref[...] = jnp.zeros_like(acc_ref)

    acc_ref[...] += jnp.sum(xT_ref[...], axis=1)

    @pl.when(i == NSTEP - 1)
    def _():
        m = acc_ref[...] * (1.0 / WINDOW)
        dn = (((0,), (0,)), ((), ()))
        feats = jax.lax.dot_general(m, Wbb_ref[...], dn,
                                    preferred_element_type=jnp.float32)
        feats = jnp.maximum(feats + bbb_ref[...], 0.0)
        Wv = Wall_ref[...].reshape(E * N_OUT, FEAT)
        dn_t = (((1,), (1,)), ((), ()))
        allh = jax.lax.dot_general(feats, Wv, dn_t,
                                   preferred_element_type=jnp.float32)
        allh_ref[...] = jnp.concatenate(
            [allh, jnp.zeros_like(allh)], axis=1)


SC_STRIPE = 128


ROW = 2 * E * N_OUT


def _sc_route_body(allh_hbm, sid_hbm, bh_hbm, out_hbm, allh_v, sid_v, bh_v,
                   out_v, nc):
    wid = lax.axis_index("s") * nc + lax.axis_index("c")
    g = wid // N_OUT
    o = wid % N_OUT
    base = g * SC_STRIPE
    pltpu.sync_copy(allh_hbm.at[pl.ds(base * ROW, SC_STRIPE * ROW)], allh_v)
    pltpu.sync_copy(sid_hbm.at[pl.ds(base, SC_STRIPE)], sid_v)
    pltpu.sync_copy(bh_hbm, bh_v)
    lanes = jax.lax.iota(jnp.int32, 16)
    for h in range(SC_STRIPE // 16):
        sidvec = sid_v[pl.ds(h * 16, 16)]
        idx = (lanes + h * 16) * ROW + sidvec * N_OUT + o
        val = plsc.load_gather(allh_v, [idx])
        bias = plsc.load_gather(bh_v, [sidvec * N_OUT + o])
        out_v[pl.ds(h * 16, 16)] = val + bias
    pltpu.sync_copy(out_v,
                    out_hbm.at[pl.ds(g * (N_OUT * SC_STRIPE) + o * SC_STRIPE,
                                     SC_STRIPE)])


@jax.jit
def kernel(x, subject_ids, W_bb, b_bb, W_heads, b_heads):
    xT = jnp.transpose(x, (1, 2, 0))
    sid = subject_ids.astype(jnp.int32)
    W_v = W_heads.transpose(0, 2, 1)
    bh_flat = b_heads.reshape(E * N_OUT)
    bbb = b_bb.reshape(1, FEAT)

    allh = pl.pallas_call(
        _tc_body,
        grid=(NSTEP,),
        in_specs=[
            pl.BlockSpec((N_CHANS, WB, B), lambda i: (0, i, 0)),
            pl.BlockSpec((N_CHANS, FEAT), lambda i: (0, 0)),
            pl.BlockSpec((1, FEAT), lambda i: (0, 0)),
            pl.BlockSpec((E, N_OUT, FEAT), lambda i: (0, 0, 0)),
        ],
        out_specs=pl.BlockSpec((B, ROW), lambda i: (0, 0)),
        out_shape=jax.ShapeDtypeStruct((B, ROW), jnp.float32),
        scratch_shapes=[pltpu.VMEM((N_CHANS, B), jnp.float32)],
    )(xT, W_bb, bbb, W_v)
    allh_flat = allh.reshape(B * ROW)

    info = plsc.get_sparse_core_info()
    nc = info.num_cores
    mesh = plsc.VectorSubcoreMesh(core_axis_name="c", subcore_axis_name="s")
    sc_route = pl.kernel(
        functools.partial(_sc_route_body, nc=nc),
        mesh=mesh,
        compiler_params=pltpu.CompilerParams(use_tc_tiling_on_sc=False,
                                             needs_layout_passes=False,
                                             skip_device_barrier=True),
        out_type=jax.ShapeDtypeStruct((N_OUT * B,), jnp.float32),
        scratch_types=[
            pltpu.VMEM((SC_STRIPE * ROW,), jnp.float32),
            pltpu.VMEM((SC_STRIPE,), jnp.int32),
            pltpu.VMEM((E * N_OUT,), jnp.float32),
            pltpu.VMEM((SC_STRIPE,), jnp.float32),
        ],
    )
    out_flat = sc_route(allh_flat, sid, bh_flat)
    return (out_flat.reshape(B // SC_STRIPE, N_OUT, SC_STRIPE)
            .transpose(1, 0, 2).reshape(N_OUT, B).T)


# speedup vs baseline: 1.6016x; 1.0051x over previous
import functools

import jax
import jax.numpy as jnp
from jax import lax
from jax.experimental import pallas as pl
from jax.experimental.pallas import tpu as pltpu
from jax.experimental.pallas import tpu_sc as plsc

B = 1024
N_CHANS = 64
WINDOW = 1000
N_OUT = 4
E = 16
FEAT = 512

WB = 40
NSTEP = WINDOW // WB


def _tc_body(xT_ref, Wbb_ref, bbb_ref, Wall_ref, allh_ref, acc_ref):
    i = pl.program_id(0)

    @pl.when(i == 0)
    def _():
        acc_ref[...] = jnp.zeros_like(acc_ref)

    acc_ref[...] += jnp.sum(xT_ref[...], axis=1)

    @pl.when(i == NSTEP - 1)
    def _():
        m = acc_ref[...] * (1.0 / WINDOW)
        dn = (((0,), (0,)), ((), ()))
        feats = jax.lax.dot_general(m, Wbb_ref[...], dn,
                                    preferred_element_type=jnp.float32)
        feats = jnp.maximum(feats + bbb_ref[...], 0.0)
        Wv = Wall_ref[...].reshape(E * N_OUT, FEAT)
        dn_t = (((1,), (1,)), ((), ()))
        allh = jax.lax.dot_general(feats, Wv, dn_t,
                                   preferred_element_type=jnp.float32)
        allh_ref[...] = jnp.concatenate(
            [allh, jnp.zeros_like(allh)], axis=1)


SC_STRIPE = 128


ROW = 2 * E * N_OUT


def _sc_route_body(allh_hbm, sid_hbm, bh_hbm, out_hbm, allh_v, sid_v, bh_v,
                   out_v, sem1, sem2, sem3, nc):
    wid = lax.axis_index("s") * nc + lax.axis_index("c")
    g = wid // N_OUT
    o = wid % N_OUT
    base = g * SC_STRIPE
    cp1 = pltpu.async_copy(allh_hbm.at[pl.ds(base * ROW, SC_STRIPE * ROW)],
                           allh_v, sem1)
    cp2 = pltpu.async_copy(sid_hbm.at[pl.ds(base, SC_STRIPE)], sid_v, sem2)
    cp3 = pltpu.async_copy(bh_hbm, bh_v, sem3)
    cp1.wait()
    cp2.wait()
    cp3.wait()
    lanes = jax.lax.iota(jnp.int32, 16)
    for h in range(SC_STRIPE // 16):
        sidvec = sid_v[pl.ds(h * 16, 16)]
        idx = (lanes + h * 16) * ROW + sidvec * N_OUT + o
        val = plsc.load_gather(allh_v, [idx])
        bias = plsc.load_gather(bh_v, [sidvec * N_OUT + o])
        out_v[pl.ds(h * 16, 16)] = val + bias
    pltpu.sync_copy(out_v,
                    out_hbm.at[pl.ds(g * (N_OUT * SC_STRIPE) + o * SC_STRIPE,
                                     SC_STRIPE)])


@jax.jit
def kernel(x, subject_ids, W_bb, b_bb, W_heads, b_heads):
    xT = jnp.transpose(x, (1, 2, 0))
    sid = subject_ids.astype(jnp.int32)
    W_v = W_heads.transpose(0, 2, 1)
    bh_flat = b_heads.reshape(E * N_OUT)
    bbb = b_bb.reshape(1, FEAT)

    allh = pl.pallas_call(
        _tc_body,
        grid=(NSTEP,),
        in_specs=[
            pl.BlockSpec((N_CHANS, WB, B), lambda i: (0, i, 0)),
            pl.BlockSpec((N_CHANS, FEAT), lambda i: (0, 0)),
            pl.BlockSpec((1, FEAT), lambda i: (0, 0)),
            pl.BlockSpec((E, N_OUT, FEAT), lambda i: (0, 0, 0)),
        ],
        out_specs=pl.BlockSpec((B, ROW), lambda i: (0, 0)),
        out_shape=jax.ShapeDtypeStruct((B, ROW), jnp.float32),
        scratch_shapes=[pltpu.VMEM((N_CHANS, B), jnp.float32)],
    )(xT, W_bb, bbb, W_v)
    allh_flat = allh.reshape(B * ROW)

    info = plsc.get_sparse_core_info()
    nc = info.num_cores
    mesh = plsc.VectorSubcoreMesh(core_axis_name="c", subcore_axis_name="s")
    sc_route = pl.kernel(
        functools.partial(_sc_route_body, nc=nc),
        mesh=mesh,
        compiler_params=pltpu.CompilerParams(use_tc_tiling_on_sc=False,
                                             needs_layout_passes=False,
                                             skip_device_barrier=True),
        out_type=jax.ShapeDtypeStruct((N_OUT * B,), jnp.float32),
        scratch_types=[
            pltpu.VMEM((SC_STRIPE * ROW,), jnp.float32),
            pltpu.VMEM((SC_STRIPE,), jnp.int32),
            pltpu.VMEM((E * N_OUT,), jnp.float32),
            pltpu.VMEM((SC_STRIPE,), jnp.float32),
            pltpu.SemaphoreType.DMA,
            pltpu.SemaphoreType.DMA,
            pltpu.SemaphoreType.DMA,
        ],
    )
    out_flat = sc_route(allh_flat, sid, bh_flat)
    return (out_flat.reshape(B // SC_STRIPE, N_OUT, SC_STRIPE)
            .transpose(1, 0, 2).reshape(N_OUT, B).T)
